# Initial kernel scaffold; baseline (speedup 1.0000x reference)
import functools

import jax
import jax.numpy as jnp
from jax import lax
from jax.experimental import pallas as pl
from jax.experimental.pallas import tpu as pltpu
from jax.experimental.pallas import tpu_sc as plsc

N_NODES = 10000
D_FEAT = 128
N_EDGES = 320000

NC = 2
NS = 16
L = 16
NW = NC * NS
EW = N_EDGES // NW
EB = 80
NB = EW // EB
DEG_R = 80
DEG_C = 128


def _newton_rsqrt(x):
  xi = plsc.bitcast(x, jnp.int32)
  yi = jnp.int32(0x5F3759DF) - (xi >> 1)
  y = plsc.bitcast(yi, jnp.float32)
  xh = x * jnp.float32(-0.5)
  for _ in range(4):
    y = y * (jnp.float32(1.5) + xh * y * y)
  return y


def _edge_body(y_hbm, src_hbm, dst_hbm, w_hbm, degp_hbm,
               src_idx, dst_idx, rows_s, rows_d, w_all, deg_priv, row_ids,
               deg_shared, sem_s, sem_d):
  cid = lax.axis_index("c")
  sid = lax.axis_index("s")
  wid = sid * NC + cid
  base = wid * EW

  pltpu.sync_copy(src_hbm.at[pl.ds(base, EW)], src_idx)
  pltpu.sync_copy(dst_hbm.at[pl.ds(base, EW)], dst_idx)

  zero16 = jnp.zeros((L,), jnp.float32)
  def _zero_row(r):
    for c in range(DEG_C // L):
      deg_priv[r, pl.ds(c * L, L)] = zero16
  pl.loop(0, DEG_R)(_zero_row)

  @pl.when(sid == 0)
  def _():
    pltpu.sync_copy(deg_priv, deg_shared)

  for g in range(DEG_R // L):
    row_ids[pl.ds(g * L, L)] = lax.iota(jnp.int32, L) + jnp.int32(g * L)

  def _block(b):
    e0 = b * EB
    cp_s = pltpu.async_copy(y_hbm.at[src_idx.at[pl.ds(e0, EB)]], rows_s, sem_s)
    cp_d = pltpu.async_copy(y_hbm.at[dst_idx.at[pl.ds(e0, EB)]], rows_d, sem_d)
    cp_s.wait()
    cp_d.wait()

    def _edge(e):
      acc = jnp.zeros((L,), jnp.float32)
      for k in range(D_FEAT // L):
        dvec = rows_s[e, pl.ds(k * L, L)] - rows_d[e, pl.ds(k * L, L)]
        acc = acc + dvec * dvec
      w_all[e0 + e] = jnp.sum(acc)
    pl.loop(0, EB)(_edge)
  pl.loop(0, NB)(_block)

  def _xform(g):
    x = w_all[pl.ds(g * L, L)] + jnp.float32(1e-7)
    y = _newton_rsqrt(x)
    w = jnp.minimum(y, jnp.float32(10.0)) + jnp.float32(1e-9)
    w = jnp.where(x > jnp.float32(400.0), jnp.float32(1e-9), w)
    w_all[pl.ds(g * L, L)] = w
  pl.loop(0, EW // L)(_xform)

  def _scatter(e):
    d = dst_idx[e]
    r = d >> 7
    c = d & 127
    deg_priv[r, c] = deg_priv[r, c] + w_all[e]
  pl.loop(0, EW)(_scatter)

  pltpu.sync_copy(w_all, w_hbm.at[pl.ds(base, EW)])

  plsc.subcore_barrier()
  pltpu.sync_copy(deg_priv, deg_shared.at[row_ids], add=True)
  plsc.subcore_barrier()

  @pl.when(sid == 0)
  def _():
    pltpu.sync_copy(deg_shared, degp_hbm.at[cid])


def _sc_attention(y, src, dst):
  mesh = plsc.VectorSubcoreMesh(core_axis_name="c", subcore_axis_name="s")
  kern = pl.kernel(
      _edge_body,
      out_type=(
          jax.ShapeDtypeStruct((N_EDGES,), jnp.float32),
          jax.ShapeDtypeStruct((NC, DEG_R, DEG_C), jnp.float32),
      ),
      mesh=mesh,
      scratch_types=[
          pltpu.VMEM((EW,), jnp.int32),
          pltpu.VMEM((EW,), jnp.int32),
          pltpu.VMEM((EB, D_FEAT), jnp.float32),
          pltpu.VMEM((EB, D_FEAT), jnp.float32),
          pltpu.VMEM((EW,), jnp.float32),
          pltpu.VMEM((DEG_R, DEG_C), jnp.float32),
          pltpu.VMEM((DEG_R,), jnp.int32),
          pltpu.VMEM_SHARED((DEG_R, DEG_C), jnp.float32),
          pltpu.SemaphoreType.DMA,
          pltpu.SemaphoreType.DMA,
      ],
  )
  return kern(y, src, dst)


def _combine_body(p_ref, o_ref):
  o_ref[...] = p_ref[0] + p_ref[1]


def _combine(degp):
  return pl.pallas_call(
      _combine_body,
      out_shape=jax.ShapeDtypeStruct((DEG_R, DEG_C), jnp.float32),
  )(degp)


def kernel(Y, edge_index):
  src = edge_index[0]
  dst = edge_index[1]
  w, degp = _sc_attention(Y, src, dst)
  deg = _combine(degp).reshape(DEG_R * DEG_C)[:N_NODES]
  return w, deg


# speedup vs baseline: 4.0355x; 4.0355x over previous
import jax
import jax.numpy as jnp
from jax import lax
from jax.experimental import pallas as pl
from jax.experimental.pallas import tpu as pltpu
from jax.experimental.pallas import tpu_sc as plsc

N_NODES = 10000
D_FEAT = 128
N_EDGES = 320000

NC = 2
NS = 16
L = 16
NW = NC * NS
EW = N_EDGES // NW
EB = 80
NB = EW // EB
DEG_N = 10240


def _newton_rsqrt(x):
  xi = plsc.bitcast(x, jnp.int32)
  yi = jnp.int32(0x5F3759DF) - (xi >> 1)
  y = plsc.bitcast(yi, jnp.float32)
  xh = x * jnp.float32(-0.5)
  for _ in range(4):
    y = y * (jnp.float32(1.5) + xh * y * y)
  return y


def _edge_body(y_hbm, src_hbm, dst_hbm, w_hbm, degp_hbm,
               src_idx, dst_idx, rows_s, rows_d, w_all, deg_priv,
               sem_s, sem_d):
  cid = lax.axis_index("c")
  sid = lax.axis_index("s")
  wid = sid * NC + cid
  base = wid * EW

  iota = lax.iota(jnp.int32, L)

  pltpu.sync_copy(src_hbm.at[pl.ds(base, EW)], src_idx)
  pltpu.sync_copy(dst_hbm.at[pl.ds(base, EW)], dst_idx)

  zero16 = jnp.zeros((L,), jnp.float32)
  @pl.loop(0, DEG_N // L)
  def _zero(i):
    deg_priv[pl.ds(i * L, L)] = zero16

  @pl.loop(0, NB)
  def _block(b):
    e0 = b * EB
    cp_s = pltpu.async_copy(y_hbm.at[src_idx.at[pl.ds(e0, EB)]], rows_s, sem_s)
    cp_d = pltpu.async_copy(y_hbm.at[dst_idx.at[pl.ds(e0, EB)]], rows_d, sem_d)
    cp_s.wait()
    cp_d.wait()

    @pl.loop(0, EB // L)
    def _group(g):
      e_ids = iota + g * L
      acc = jnp.zeros((L,), jnp.float32)
      for k in range(D_FEAT):
        kv = jnp.full((L,), k, jnp.int32)
        s = plsc.load_gather(rows_s, [e_ids, kv])
        d = plsc.load_gather(rows_d, [e_ids, kv])
        dv = s - d
        acc = acc + dv * dv
      w_all[pl.ds(e0 + g * L, L)] = acc

  rot = (iota + jnp.int32(L - 1)) & jnp.int32(L - 1)
  last_lane = iota == jnp.int32(L - 1)

  @pl.loop(0, EW // L)
  def _xform(g):
    off = g * L
    x = w_all[pl.ds(off, L)] + jnp.float32(1e-7)
    y = _newton_rsqrt(x)
    w = jnp.minimum(y, jnp.float32(10.0)) + jnp.float32(1e-9)
    w = jnp.where(x > jnp.float32(400.0), jnp.float32(1e-9), w)
    w_all[pl.ds(off, L)] = w

    d16 = dst_idx[pl.ds(off, L)]
    k, v = plsc.sort_key_val(d16, w)
    c = plsc.cumsum(v)
    _, k_next = plsc.sort_key_val(rot, k)
    neq = k != k_next
    is_end = neq | last_lane
    m2 = neq & jnp.logical_not(last_lane)
    plsc.addupdate_scatter(deg_priv, [k], c, mask=is_end)
    plsc.addupdate_scatter(deg_priv, [k_next], -c, mask=m2)

  pltpu.sync_copy(w_all, w_hbm.at[pl.ds(base, EW)])
  pltpu.sync_copy(deg_priv, degp_hbm.at[wid])


def _sc_attention(y, src, dst):
  mesh = plsc.VectorSubcoreMesh(core_axis_name="c", subcore_axis_name="s")
  kern = pl.kernel(
      _edge_body,
      out_type=(
          jax.ShapeDtypeStruct((N_EDGES,), jnp.float32),
          jax.ShapeDtypeStruct((NW, DEG_N), jnp.float32),
      ),
      mesh=mesh,
      scratch_types=[
          pltpu.VMEM((EW,), jnp.int32),
          pltpu.VMEM((EW,), jnp.int32),
          pltpu.VMEM((EB, D_FEAT), jnp.float32),
          pltpu.VMEM((EB, D_FEAT), jnp.float32),
          pltpu.VMEM((EW,), jnp.float32),
          pltpu.VMEM((DEG_N,), jnp.float32),
          pltpu.SemaphoreType.DMA,
          pltpu.SemaphoreType.DMA,
      ],
      compiler_params=pltpu.CompilerParams(needs_layout_passes=False),
  )
  return kern(y, src, dst)


def _combine_body(p_ref, o_ref):
  o_ref[...] = jnp.sum(p_ref[...], axis=0)


def _combine(degp):
  return pl.pallas_call(
      _combine_body,
      out_shape=jax.ShapeDtypeStruct((DEG_N // 128, 128), jnp.float32),
  )(degp)


def kernel(Y, edge_index):
  src = edge_index[0]
  dst = edge_index[1]
  w, degp = _sc_attention(Y, src, dst)
  deg = _combine(degp.reshape(NW, DEG_N // 128, 128)).reshape(DEG_N)[:N_NODES]
  return w, deg


# speedup vs baseline: 12.0701x; 2.9910x over previous
import jax
import jax.numpy as jnp
from jax import lax
from jax.experimental import pallas as pl
from jax.experimental.pallas import tpu as pltpu
from jax.experimental.pallas import tpu_sc as plsc

N_NODES = 10000
D_FEAT = 128
N_EDGES = 320000

NC = 2
NS = 16
L = 16
NW = NC * NS
EW = N_EDGES // NW
EB = 80
NB = EW // EB
DEG_N = 10240


def _newton_rsqrt(x):
  xi = plsc.bitcast(x, jnp.int32)
  yi = jnp.int32(0x5F3759DF) - (xi >> 1)
  y = plsc.bitcast(yi, jnp.float32)
  xh = x * jnp.float32(-0.5)
  for _ in range(4):
    y = y * (jnp.float32(1.5) + xh * y * y)
  return y


def _edge_body(y_hbm, src_hbm, dst_hbm, w_hbm, degp_hbm,
               src_idx, dst_idx, rows_s, rows_d, w_all, deg_priv,
               sem_s, sem_d):
  cid = lax.axis_index("c")
  sid = lax.axis_index("s")
  wid = sid * NC + cid
  base = wid * EW

  iota = lax.iota(jnp.int32, L)

  pltpu.sync_copy(src_hbm.at[pl.ds(base, EW)], src_idx)
  pltpu.sync_copy(dst_hbm.at[pl.ds(base, EW)], dst_idx)

  zero16 = jnp.zeros((L,), jnp.float32)
  @pl.loop(0, DEG_N // L)
  def _zero(i):
    deg_priv[pl.ds(i * L, L)] = zero16

  @pl.loop(0, NB)
  def _block(b):
    e0 = b * EB
    cp_s = pltpu.async_copy(y_hbm.at[src_idx.at[pl.ds(e0, EB)]], rows_s, sem_s)
    cp_d = pltpu.async_copy(y_hbm.at[dst_idx.at[pl.ds(e0, EB)]], rows_d, sem_d)
    cp_s.wait()
    cp_d.wait()

    @pl.loop(0, EB // L)
    def _group(g):
      gbase = g * L
      w16 = zero16
      for e in range(L):
        erow = gbase + e
        acc = zero16
        for k in range(D_FEAT // L):
          dv = rows_s[erow, pl.ds(k * L, L)] - rows_d[erow, pl.ds(k * L, L)]
          acc = acc + dv * dv
        w16 = jnp.where(iota == e, jnp.sum(acc), w16)
      w_all[pl.ds(e0 + gbase, L)] = w16

  rot = (iota + jnp.int32(L - 1)) & jnp.int32(L - 1)
  last_lane = iota == jnp.int32(L - 1)

  @pl.loop(0, EW // L)
  def _xform(g):
    off = g * L
    x = w_all[pl.ds(off, L)] + jnp.float32(1e-7)
    y = _newton_rsqrt(x)
    w = jnp.minimum(y, jnp.float32(10.0)) + jnp.float32(1e-9)
    w = jnp.where(x > jnp.float32(400.0), jnp.float32(1e-9), w)
    w_all[pl.ds(off, L)] = w

    d16 = dst_idx[pl.ds(off, L)]
    k, v = plsc.sort_key_val(d16, w)
    c = plsc.cumsum(v)
    _, k_next = plsc.sort_key_val(rot, k)
    neq = k != k_next
    is_end = neq | last_lane
    m2 = neq & jnp.logical_not(last_lane)
    plsc.addupdate_scatter(deg_priv, [k], c, mask=is_end)
    plsc.addupdate_scatter(deg_priv, [k_next], -c, mask=m2)

  pltpu.sync_copy(w_all, w_hbm.at[pl.ds(base, EW)])
  pltpu.sync_copy(deg_priv, degp_hbm.at[wid])


def _sc_attention(y, src, dst):
  mesh = plsc.VectorSubcoreMesh(core_axis_name="c", subcore_axis_name="s")
  kern = pl.kernel(
      _edge_body,
      out_type=(
          jax.ShapeDtypeStruct((N_EDGES,), jnp.float32),
          jax.ShapeDtypeStruct((NW, DEG_N), jnp.float32),
      ),
      mesh=mesh,
      scratch_types=[
          pltpu.VMEM((EW,), jnp.int32),
          pltpu.VMEM((EW,), jnp.int32),
          pltpu.VMEM((EB, D_FEAT), jnp.float32),
          pltpu.VMEM((EB, D_FEAT), jnp.float32),
          pltpu.VMEM((EW,), jnp.float32),
          pltpu.VMEM((DEG_N,), jnp.float32),
          pltpu.SemaphoreType.DMA,
          pltpu.SemaphoreType.DMA,
      ],
      compiler_params=pltpu.CompilerParams(needs_layout_passes=False),
  )
  return kern(y, src, dst)


def _combine_body(p_ref, o_ref):
  o_ref[...] = jnp.sum(p_ref[...], axis=0)


def _combine(degp):
  return pl.pallas_call(
      _combine_body,
      out_shape=jax.ShapeDtypeStruct((DEG_N // 128, 128), jnp.float32),
  )(degp)


def kernel(Y, edge_index):
  src = edge_index[0]
  dst = edge_index[1]
  w, degp = _sc_attention(Y, src, dst)
  deg = _combine(degp.reshape(NW, DEG_N // 128, 128)).reshape(DEG_N)[:N_NODES]
  return w, deg
